# initial kernel scaffold (unmeasured)
import jax
import jax.numpy as jnp
from jax import lax
from jax.experimental import pallas as pl
from jax.experimental.pallas import tpu as pltpu

N_DEV = 8
M = 2048
N = 2048
H_SHARD = 4096
H_CHUNK = 1024
CHUNK_ROWS = M // N_DEV


def _mlp_body(x_ref, w1_ref, w2_ref, out_ref):
    k = pl.program_id(0)
    h = jnp.dot(x_ref[...], w1_ref[...], preferred_element_type=jnp.bfloat16)
    h = jnp.maximum(h, 0.0)
    p = jnp.dot(h, w2_ref[...], preferred_element_type=jnp.float32)

    @pl.when(k == 0)
    def _():
        out_ref[...] = p

    @pl.when(k != 0)
    def _():
        out_ref[...] = out_ref[...] + p


def _allreduce_body(p_ref, out_ref, recv_buf, acc_ref, send_sems, recv_sems):
    my = lax.axis_index("i")
    left = jnp.mod(my - 1, N_DEV)
    right = jnp.mod(my + 1, N_DEV)

    barrier_sem = pltpu.get_barrier_semaphore()
    for nbr in (left, right):
        pltpu.semaphore_signal(
            barrier_sem, inc=1,
            device_id=(nbr,), device_id_type=pltpu.DeviceIdType.MESH,
        )
    pltpu.semaphore_wait(barrier_sem, 2)

    def chunk(ref, c):
        return ref.at[pl.ds(c * CHUNK_ROWS, CHUNK_ROWS), :]

    for s in range(N_DEV - 1):
        send_c = jnp.mod(my - s, N_DEV)
        recv_c = jnp.mod(my - s - 1, N_DEV)
        src = chunk(p_ref, send_c) if s == 0 else acc_ref
        rdma = pltpu.make_async_remote_copy(
            src_ref=src,
            dst_ref=recv_buf.at[s],
            send_sem=send_sems.at[s],
            recv_sem=recv_sems.at[s],
            device_id=(right,),
            device_id_type=pltpu.DeviceIdType.MESH,
        )
        rdma.start()
        rdma.wait()
        acc_ref[...] = recv_buf[s] + p_ref[pl.ds(recv_c * CHUNK_ROWS, CHUNK_ROWS), :]

    own_c = jnp.mod(my + 1, N_DEV)
    out_ref[pl.ds(own_c * CHUNK_ROWS, CHUNK_ROWS), :] = acc_ref[...]

    for t in range(N_DEV - 1):
        k = (N_DEV - 1) + t
        send_c = jnp.mod(my + 1 - t, N_DEV)
        rdma = pltpu.make_async_remote_copy(
            src_ref=chunk(out_ref, send_c),
            dst_ref=chunk(out_ref, send_c),
            send_sem=send_sems.at[k],
            recv_sem=recv_sems.at[k],
            device_id=(right,),
            device_id_type=pltpu.DeviceIdType.MESH,
        )
        rdma.start()
        rdma.wait()


def kernel(x, W1, W2):
    xb = x.astype(jnp.bfloat16)
    w1b = W1.astype(jnp.bfloat16)
    w2b = W2.astype(jnp.bfloat16)

    partial = pl.pallas_call(
        _mlp_body,
        grid=(H_SHARD // H_CHUNK,),
        in_specs=[
            pl.BlockSpec((M, 2048), lambda k: (0, 0)),
            pl.BlockSpec((2048, H_CHUNK), lambda k: (0, k)),
            pl.BlockSpec((H_CHUNK, N), lambda k: (k, 0)),
        ],
        out_specs=pl.BlockSpec((M, N), lambda k: (0, 0)),
        out_shape=jax.ShapeDtypeStruct((M, N), jnp.float32),
    )(xb, w1b, w2b)

    return pl.pallas_call(
        _allreduce_body,
        out_shape=jax.ShapeDtypeStruct((M, N), jnp.float32),
        in_specs=[pl.BlockSpec(memory_space=pltpu.VMEM)],
        out_specs=pl.BlockSpec(memory_space=pltpu.VMEM),
        scratch_shapes=[
            pltpu.VMEM((N_DEV - 1, CHUNK_ROWS, N), jnp.float32),
            pltpu.VMEM((CHUNK_ROWS, N), jnp.float32),
            pltpu.SemaphoreType.DMA((2 * (N_DEV - 1),)),
            pltpu.SemaphoreType.DMA((2 * (N_DEV - 1),)),
        ],
        compiler_params=pltpu.CompilerParams(collective_id=0),
    )(partial)


# baseline (device time: 502040 ns/iter reference)
import jax
import jax.numpy as jnp
from jax import lax
from jax.experimental import pallas as pl
from jax.experimental.pallas import tpu as pltpu

N_DEV = 8
M = 2048
N = 2048
H_SHARD = 4096
H_CHUNK = 1024
CHUNK_ROWS = M // N_DEV


def _mlp_body(x_ref, w1_ref, w2_ref, out_ref):
    k = pl.program_id(0)
    h = jnp.dot(x_ref[...], w1_ref[...], preferred_element_type=jnp.float32)
    h = jnp.maximum(h, 0.0).astype(jnp.bfloat16)
    p = jnp.dot(h, w2_ref[...], preferred_element_type=jnp.float32)

    @pl.when(k == 0)
    def _():
        out_ref[...] = p

    @pl.when(k != 0)
    def _():
        out_ref[...] = out_ref[...] + p


def _allreduce_body(p_ref, out_ref, recv_buf, acc_ref, send_sems, recv_sems):
    my = lax.axis_index("i")
    left = jnp.mod(my - 1, N_DEV)
    right = jnp.mod(my + 1, N_DEV)

    barrier_sem = pltpu.get_barrier_semaphore()
    for nbr in (left, right):
        pltpu.semaphore_signal(
            barrier_sem, inc=1,
            device_id=(nbr,), device_id_type=pltpu.DeviceIdType.MESH,
        )
    pltpu.semaphore_wait(barrier_sem, 2)

    def chunk(ref, c):
        return ref.at[pl.ds(c * CHUNK_ROWS, CHUNK_ROWS), :]

    for s in range(N_DEV - 1):
        send_c = jnp.mod(my - s, N_DEV)
        recv_c = jnp.mod(my - s - 1, N_DEV)
        src = chunk(p_ref, send_c) if s == 0 else acc_ref
        rdma = pltpu.make_async_remote_copy(
            src_ref=src,
            dst_ref=recv_buf.at[s],
            send_sem=send_sems.at[s],
            recv_sem=recv_sems.at[s],
            device_id=(right,),
            device_id_type=pltpu.DeviceIdType.MESH,
        )
        rdma.start()
        rdma.wait()
        acc_ref[...] = recv_buf[s] + p_ref[pl.ds(recv_c * CHUNK_ROWS, CHUNK_ROWS), :]

    own_c = jnp.mod(my + 1, N_DEV)
    out_ref[pl.ds(own_c * CHUNK_ROWS, CHUNK_ROWS), :] = acc_ref[...]

    for t in range(N_DEV - 1):
        k = (N_DEV - 1) + t
        send_c = jnp.mod(my + 1 - t, N_DEV)
        rdma = pltpu.make_async_remote_copy(
            src_ref=chunk(out_ref, send_c),
            dst_ref=chunk(out_ref, send_c),
            send_sem=send_sems.at[k],
            recv_sem=recv_sems.at[k],
            device_id=(right,),
            device_id_type=pltpu.DeviceIdType.MESH,
        )
        rdma.start()
        rdma.wait()


def kernel(x, W1, W2):
    xb = x.astype(jnp.bfloat16)
    w1b = W1.astype(jnp.bfloat16)
    w2b = W2.astype(jnp.bfloat16)

    partial = pl.pallas_call(
        _mlp_body,
        grid=(H_SHARD // H_CHUNK,),
        in_specs=[
            pl.BlockSpec((M, 2048), lambda k: (0, 0)),
            pl.BlockSpec((2048, H_CHUNK), lambda k: (0, k)),
            pl.BlockSpec((H_CHUNK, N), lambda k: (k, 0)),
        ],
        out_specs=pl.BlockSpec((M, N), lambda k: (0, 0)),
        out_shape=jax.ShapeDtypeStruct((M, N), jnp.float32),
        compiler_params=pltpu.CompilerParams(
            vmem_limit_bytes=60 * 1024 * 1024,
        ),
    )(xb, w1b, w2b)

    return pl.pallas_call(
        _allreduce_body,
        out_shape=jax.ShapeDtypeStruct((M, N), jnp.float32),
        in_specs=[pl.BlockSpec(memory_space=pltpu.VMEM)],
        out_specs=pl.BlockSpec(memory_space=pltpu.VMEM),
        scratch_shapes=[
            pltpu.VMEM((N_DEV - 1, CHUNK_ROWS, N), jnp.float32),
            pltpu.VMEM((CHUNK_ROWS, N), jnp.float32),
            pltpu.SemaphoreType.DMA((2 * (N_DEV - 1),)),
            pltpu.SemaphoreType.DMA((2 * (N_DEV - 1),)),
        ],
        compiler_params=pltpu.CompilerParams(
            collective_id=0,
            vmem_limit_bytes=60 * 1024 * 1024,
        ),
    )(partial)


# device time: 265776 ns/iter; 1.8890x vs baseline; 1.8890x over previous
import jax
import jax.numpy as jnp
from jax import lax
from jax.experimental import pallas as pl
from jax.experimental.pallas import tpu as pltpu

N_DEV = 8
M = 2048
N = 2048
H_SHARD = 4096
H_CHUNK = 1024
CHUNK_ROWS = M // N_DEV
HALF_COLS = N // 2
N_STEPS = N_DEV - 1


def _mlp_body(x_ref, w1_ref, w2_ref, out_ref):
    k = pl.program_id(0)
    h = jnp.dot(x_ref[...], w1_ref[...], preferred_element_type=jnp.float32)
    h = jnp.maximum(h, 0.0).astype(jnp.bfloat16)
    p = jnp.dot(h, w2_ref[...], preferred_element_type=jnp.float32)

    @pl.when(k == 0)
    def _():
        out_ref[...] = p

    @pl.when(k != 0)
    def _():
        out_ref[...] = out_ref[...] + p


def _ring_pos(j):
    return jnp.where(j < 4, j, 11 - j)


def _allreduce_body(p_ref, out_ref, gath_ref, recv_buf, send_buf,
                    send_sems, recv_sems):
    my = lax.axis_index("i")
    r = _ring_pos(my)
    nxt = _ring_pos(jnp.mod(r + 1, N_DEV))
    prv = _ring_pos(jnp.mod(r - 1, N_DEV))
    rr = jnp.mod(N_DEV - r, N_DEV)
    rd = (r, rr)
    tgt = (nxt, prv)

    barrier_sem = pltpu.get_barrier_semaphore()
    for nbr in (nxt, prv):
        pltpu.semaphore_signal(
            barrier_sem, inc=1,
            device_id=(nbr,), device_id_type=pltpu.DeviceIdType.MESH,
        )
    pltpu.semaphore_wait(barrier_sem, 2)

    def p_piece(c, d):
        return p_ref[pl.ds(c * CHUNK_ROWS, CHUNK_ROWS),
                     pl.ds(d * HALF_COLS, HALF_COLS)]

    def gath_piece(c, d):
        return gath_ref.at[pl.ds(c * CHUNK_ROWS, CHUNK_ROWS),
                           pl.ds(d * HALF_COLS, HALF_COLS)]

    for s in range(N_STEPS):
        rdmas = []
        for d in range(2):
            if s == 0:
                send_c = jnp.mod(rd[d], N_DEV)
                send_buf[d] = p_piece(send_c, d).astype(jnp.bfloat16)
            rdma = pltpu.make_async_remote_copy(
                src_ref=send_buf.at[d],
                dst_ref=recv_buf.at[d, s],
                send_sem=send_sems.at[d, s],
                recv_sem=recv_sems.at[d, s],
                device_id=(tgt[d],),
                device_id_type=pltpu.DeviceIdType.MESH,
            )
            rdma.start()
            rdmas.append(rdma)
        for d in range(2):
            rdmas[d].wait()
            recv_c = jnp.mod(rd[d] - s - 1, N_DEV)
            val = recv_buf[d, s].astype(jnp.float32) + p_piece(recv_c, d)
            if s < N_STEPS - 1:
                send_buf[d] = val.astype(jnp.bfloat16)
            else:
                own_c = jnp.mod(rd[d] + 1, N_DEV)
                out_ref[pl.ds(own_c * CHUNK_ROWS, CHUNK_ROWS),
                        pl.ds(d * HALF_COLS, HALF_COLS)] = val
                gath_piece(own_c, d)[...] = val.astype(jnp.bfloat16)

    for t in range(N_STEPS):
        k = N_STEPS + t
        rdmas = []
        for d in range(2):
            send_c = jnp.mod(rd[d] + 1 - t, N_DEV)
            rdma = pltpu.make_async_remote_copy(
                src_ref=gath_piece(send_c, d),
                dst_ref=gath_piece(send_c, d),
                send_sem=send_sems.at[d, k],
                recv_sem=recv_sems.at[d, k],
                device_id=(tgt[d],),
                device_id_type=pltpu.DeviceIdType.MESH,
            )
            rdma.start()
            rdmas.append(rdma)
        for d in range(2):
            rdmas[d].wait()
            recv_c = jnp.mod(rd[d] - t, N_DEV)
            out_ref[pl.ds(recv_c * CHUNK_ROWS, CHUNK_ROWS),
                    pl.ds(d * HALF_COLS, HALF_COLS)] = (
                gath_ref[pl.ds(recv_c * CHUNK_ROWS, CHUNK_ROWS),
                         pl.ds(d * HALF_COLS, HALF_COLS)].astype(jnp.float32))


def kernel(x, W1, W2):
    xb = x.astype(jnp.bfloat16)
    w1b = W1.astype(jnp.bfloat16)
    w2b = W2.astype(jnp.bfloat16)

    partial = pl.pallas_call(
        _mlp_body,
        grid=(H_SHARD // H_CHUNK,),
        in_specs=[
            pl.BlockSpec((M, 2048), lambda k: (0, 0)),
            pl.BlockSpec((2048, H_CHUNK), lambda k: (0, k)),
            pl.BlockSpec((H_CHUNK, N), lambda k: (k, 0)),
        ],
        out_specs=pl.BlockSpec((M, N), lambda k: (0, 0)),
        out_shape=jax.ShapeDtypeStruct((M, N), jnp.float32),
        compiler_params=pltpu.CompilerParams(
            vmem_limit_bytes=60 * 1024 * 1024,
        ),
    )(xb, w1b, w2b)

    return pl.pallas_call(
        _allreduce_body,
        out_shape=jax.ShapeDtypeStruct((M, N), jnp.float32),
        in_specs=[pl.BlockSpec(memory_space=pltpu.VMEM)],
        out_specs=pl.BlockSpec(memory_space=pltpu.VMEM),
        scratch_shapes=[
            pltpu.VMEM((M, N), jnp.bfloat16),
            pltpu.VMEM((2, N_STEPS, CHUNK_ROWS, HALF_COLS), jnp.bfloat16),
            pltpu.VMEM((2, CHUNK_ROWS, HALF_COLS), jnp.bfloat16),
            pltpu.SemaphoreType.DMA((2, 2 * N_STEPS)),
            pltpu.SemaphoreType.DMA((2, 2 * N_STEPS)),
        ],
        compiler_params=pltpu.CompilerParams(
            collective_id=0,
            vmem_limit_bytes=60 * 1024 * 1024,
        ),
    )(partial)


# device time: 241771 ns/iter; 2.0765x vs baseline; 1.0993x over previous
import jax
import jax.numpy as jnp
from jax import lax
from jax.experimental import pallas as pl
from jax.experimental.pallas import tpu as pltpu

N_DEV = 8
M = 2048
N = 2048
H_SHARD = 4096
H_CHUNK = 512
CHUNK_ROWS = M // N_DEV
HALF_COLS = N // 2
N_STEPS = N_DEV - 1


def _mlp_body(x_ref, w1_ref, w2_ref, out_ref):
    k = pl.program_id(0)
    h = jnp.dot(x_ref[...], w1_ref[...].astype(jnp.bfloat16),
                preferred_element_type=jnp.float32)
    h = jnp.maximum(h, 0.0).astype(jnp.bfloat16)
    p = jnp.dot(h, w2_ref[...].astype(jnp.bfloat16),
                preferred_element_type=jnp.float32)

    @pl.when(k == 0)
    def _():
        out_ref[...] = p

    @pl.when(k != 0)
    def _():
        out_ref[...] = out_ref[...] + p


def _ring_pos(j):
    return jnp.where(j < 4, j, 11 - j)


def _allreduce_body(p_ref, out_ref, gath_ref, recv_buf, send_buf,
                    send_sems, recv_sems):
    my = lax.axis_index("i")
    r = _ring_pos(my)
    nxt = _ring_pos(jnp.mod(r + 1, N_DEV))
    prv = _ring_pos(jnp.mod(r - 1, N_DEV))
    rr = jnp.mod(N_DEV - r, N_DEV)
    rd = (r, rr)
    tgt = (nxt, prv)

    barrier_sem = pltpu.get_barrier_semaphore()
    for nbr in (nxt, prv):
        pltpu.semaphore_signal(
            barrier_sem, inc=1,
            device_id=(nbr,), device_id_type=pltpu.DeviceIdType.MESH,
        )
    pltpu.semaphore_wait(barrier_sem, 2)

    def p_piece(c, d):
        return p_ref[pl.ds(c * CHUNK_ROWS, CHUNK_ROWS),
                     pl.ds(d * HALF_COLS, HALF_COLS)]

    def gath_piece(c, d):
        return gath_ref.at[pl.ds(c * CHUNK_ROWS, CHUNK_ROWS),
                           pl.ds(d * HALF_COLS, HALF_COLS)]

    for s in range(N_STEPS):
        rdmas = []
        for d in range(2):
            if s == 0:
                send_c = jnp.mod(rd[d], N_DEV)
                send_buf[d] = p_piece(send_c, d).astype(jnp.bfloat16)
            rdma = pltpu.make_async_remote_copy(
                src_ref=send_buf.at[d],
                dst_ref=recv_buf.at[d, s],
                send_sem=send_sems.at[d, s],
                recv_sem=recv_sems.at[d, s],
                device_id=(tgt[d],),
                device_id_type=pltpu.DeviceIdType.MESH,
            )
            rdma.start()
            rdmas.append(rdma)
        for d in range(2):
            rdmas[d].wait()
            recv_c = jnp.mod(rd[d] - s - 1, N_DEV)
            val = recv_buf[d, s].astype(jnp.float32) + p_piece(recv_c, d)
            if s < N_STEPS - 1:
                send_buf[d] = val.astype(jnp.bfloat16)
            else:
                own_c = jnp.mod(rd[d] + 1, N_DEV)
                out_ref[pl.ds(own_c * CHUNK_ROWS, CHUNK_ROWS),
                        pl.ds(d * HALF_COLS, HALF_COLS)] = val
                gath_piece(own_c, d)[...] = val.astype(jnp.bfloat16)

    for t in range(N_STEPS):
        k = N_STEPS + t
        rdmas = []
        for d in range(2):
            send_c = jnp.mod(rd[d] + 1 - t, N_DEV)
            rdma = pltpu.make_async_remote_copy(
                src_ref=gath_piece(send_c, d),
                dst_ref=gath_piece(send_c, d),
                send_sem=send_sems.at[d, k],
                recv_sem=recv_sems.at[d, k],
                device_id=(tgt[d],),
                device_id_type=pltpu.DeviceIdType.MESH,
            )
            rdma.start()
            rdmas.append(rdma)
        for d in range(2):
            rdmas[d].wait()
            recv_c = jnp.mod(rd[d] - t, N_DEV)
            out_ref[pl.ds(recv_c * CHUNK_ROWS, CHUNK_ROWS),
                    pl.ds(d * HALF_COLS, HALF_COLS)] = (
                gath_ref[pl.ds(recv_c * CHUNK_ROWS, CHUNK_ROWS),
                         pl.ds(d * HALF_COLS, HALF_COLS)].astype(jnp.float32))


def kernel(x, W1, W2):
    xb = x.astype(jnp.bfloat16)
    partial = pl.pallas_call(
        _mlp_body,
        grid=(H_SHARD // H_CHUNK,),
        in_specs=[
            pl.BlockSpec((M, 2048), lambda k: (0, 0)),
            pl.BlockSpec((2048, H_CHUNK), lambda k: (0, k)),
            pl.BlockSpec((H_CHUNK, N), lambda k: (k, 0)),
        ],
        out_specs=pl.BlockSpec((M, N), lambda k: (0, 0)),
        out_shape=jax.ShapeDtypeStruct((M, N), jnp.float32),
        compiler_params=pltpu.CompilerParams(
            vmem_limit_bytes=60 * 1024 * 1024,
        ),
    )(xb, W1, W2)

    return pl.pallas_call(
        _allreduce_body,
        out_shape=jax.ShapeDtypeStruct((M, N), jnp.float32),
        in_specs=[pl.BlockSpec(memory_space=pltpu.VMEM)],
        out_specs=pl.BlockSpec(memory_space=pltpu.VMEM),
        scratch_shapes=[
            pltpu.VMEM((M, N), jnp.bfloat16),
            pltpu.VMEM((2, N_STEPS, CHUNK_ROWS, HALF_COLS), jnp.bfloat16),
            pltpu.VMEM((2, CHUNK_ROWS, HALF_COLS), jnp.bfloat16),
            pltpu.SemaphoreType.DMA((2, 2 * N_STEPS)),
            pltpu.SemaphoreType.DMA((2, 2 * N_STEPS)),
        ],
        compiler_params=pltpu.CompilerParams(
            collective_id=0,
            vmem_limit_bytes=60 * 1024 * 1024,
        ),
    )(partial)


# device time: 219527 ns/iter; 2.2869x vs baseline; 1.1013x over previous
import jax
import jax.numpy as jnp
from jax import lax
from jax.experimental import pallas as pl
from jax.experimental.pallas import tpu as pltpu

N_DEV = 8
M = 2048
N = 2048
H_SHARD = 4096
H_CHUNK = 512
CHUNK_ROWS = M // N_DEV
Q_COLS = N // 4
N_STEPS = N_DEV - 1


def _mlp_body(x_ref, w1_ref, w2_ref, out_ref):
    k = pl.program_id(0)
    h = jnp.dot(x_ref[...], w1_ref[...].astype(jnp.bfloat16),
                preferred_element_type=jnp.float32)
    h = jnp.maximum(h, 0.0).astype(jnp.bfloat16)
    p = jnp.dot(h, w2_ref[...].astype(jnp.bfloat16),
                preferred_element_type=jnp.float32)

    @pl.when(k == 0)
    def _():
        out_ref[...] = p

    @pl.when(k != 0)
    def _():
        out_ref[...] = out_ref[...] + p


def _ring_pos(j):
    return jnp.where(j < 4, j, 11 - j)


def _allreduce_body(p_ref, out_ref, gath_ref, recv_buf, send_buf,
                    send_sems, recv_sems):
    my = lax.axis_index("i")
    r = _ring_pos(my)
    nxt = _ring_pos(jnp.mod(r + 1, N_DEV))
    prv = _ring_pos(jnp.mod(r - 1, N_DEV))
    rr = jnp.mod(N_DEV - r, N_DEV)

    N_Q = 4
    q_rd = [r, r, rr, rr]
    q_tgt = [nxt, nxt, prv, prv]

    barrier_sem = pltpu.get_barrier_semaphore()
    for nbr in (nxt, prv):
        pltpu.semaphore_signal(
            barrier_sem, inc=1,
            device_id=(nbr,), device_id_type=pltpu.DeviceIdType.MESH,
        )
    pltpu.semaphore_wait(barrier_sem, 2)

    def p_piece(c, q):
        return p_ref[pl.ds(c * CHUNK_ROWS, CHUNK_ROWS),
                     pl.ds(q * Q_COLS, Q_COLS)]

    def gath_piece(c, q):
        return gath_ref.at[pl.ds(c * CHUNK_ROWS, CHUNK_ROWS),
                           pl.ds(q * Q_COLS, Q_COLS)]

    def start_rs(q, s):
        rdma = pltpu.make_async_remote_copy(
            src_ref=send_buf.at[q],
            dst_ref=recv_buf.at[q, s],
            send_sem=send_sems.at[q, s],
            recv_sem=recv_sems.at[q, s],
            device_id=(q_tgt[q],),
            device_id_type=pltpu.DeviceIdType.MESH,
        )
        rdma.start()
        return rdma

    def start_ag(q, h):
        send_c = jnp.mod(q_rd[q] + 1 - (h - N_STEPS), N_DEV)
        rdma = pltpu.make_async_remote_copy(
            src_ref=gath_piece(send_c, q),
            dst_ref=gath_piece(send_c, q),
            send_sem=send_sems.at[q, h],
            recv_sem=recv_sems.at[q, h],
            device_id=(q_tgt[q],),
            device_id_type=pltpu.DeviceIdType.MESH,
        )
        rdma.start()
        return rdma

    inflight = []
    for q in range(N_Q):
        send_buf[q] = p_piece(jnp.mod(q_rd[q], N_DEV), q).astype(jnp.bfloat16)
        inflight.append(start_rs(q, 0))

    for h in range(2 * N_STEPS):
        for q in range(N_Q):
            inflight[q].wait()
            if h < N_STEPS - 1:
                recv_c = jnp.mod(q_rd[q] - h - 1, N_DEV)
                val = recv_buf[q, h].astype(jnp.float32) + p_piece(recv_c, q)
                send_buf[q] = val.astype(jnp.bfloat16)
                inflight[q] = start_rs(q, h + 1)
            elif h == N_STEPS - 1:
                own_c = jnp.mod(q_rd[q] + 1, N_DEV)
                val = recv_buf[q, h].astype(jnp.float32) + p_piece(own_c, q)
                gath_piece(own_c, q)[...] = val.astype(jnp.bfloat16)
                inflight[q] = start_ag(q, h + 1)
                out_ref[pl.ds(own_c * CHUNK_ROWS, CHUNK_ROWS),
                        pl.ds(q * Q_COLS, Q_COLS)] = val
            else:
                if h < 2 * N_STEPS - 1:
                    inflight[q] = start_ag(q, h + 1)
                recv_c = jnp.mod(q_rd[q] - (h - N_STEPS), N_DEV)
                out_ref[pl.ds(recv_c * CHUNK_ROWS, CHUNK_ROWS),
                        pl.ds(q * Q_COLS, Q_COLS)] = (
                    gath_ref[pl.ds(recv_c * CHUNK_ROWS, CHUNK_ROWS),
                             pl.ds(q * Q_COLS, Q_COLS)].astype(jnp.float32))


def kernel(x, W1, W2):
    xb = x.astype(jnp.bfloat16)
    partial = pl.pallas_call(
        _mlp_body,
        grid=(H_SHARD // H_CHUNK,),
        in_specs=[
            pl.BlockSpec((M, 2048), lambda k: (0, 0)),
            pl.BlockSpec((2048, H_CHUNK), lambda k: (0, k)),
            pl.BlockSpec((H_CHUNK, N), lambda k: (k, 0)),
        ],
        out_specs=pl.BlockSpec((M, N), lambda k: (0, 0)),
        out_shape=jax.ShapeDtypeStruct((M, N), jnp.float32),
        compiler_params=pltpu.CompilerParams(
            vmem_limit_bytes=60 * 1024 * 1024,
        ),
    )(xb, W1, W2)

    return pl.pallas_call(
        _allreduce_body,
        out_shape=jax.ShapeDtypeStruct((M, N), jnp.float32),
        in_specs=[pl.BlockSpec(memory_space=pltpu.VMEM)],
        out_specs=pl.BlockSpec(memory_space=pltpu.VMEM),
        scratch_shapes=[
            pltpu.VMEM((M, N), jnp.bfloat16),
            pltpu.VMEM((4, N_STEPS, CHUNK_ROWS, Q_COLS), jnp.bfloat16),
            pltpu.VMEM((4, CHUNK_ROWS, Q_COLS), jnp.bfloat16),
            pltpu.SemaphoreType.DMA((4, 2 * N_STEPS)),
            pltpu.SemaphoreType.DMA((4, 2 * N_STEPS)),
        ],
        compiler_params=pltpu.CompilerParams(
            collective_id=0,
            vmem_limit_bytes=60 * 1024 * 1024,
        ),
    )(partial)
